# Initial kernel scaffold; baseline (speedup 1.0000x reference)
import jax
import jax.numpy as jnp
from jax.experimental import pallas as pl


def kernel(x, edge_index, edge_attr, W1, W2, b, gamma, beta):
    raise NotImplementedError("write your pallas kernel here")


# speedup vs baseline: 4.1765x; 4.1765x over previous
import functools

import jax
import jax.numpy as jnp
from jax import lax
from jax.experimental import pallas as pl
from jax.experimental.pallas import tpu as pltpu
from jax.experimental.pallas import tpu_sc as plsc

_N = 10000
_E = 320000
_D = 128
_DE = 16

_BE = 8000


def _edge_gate_body(ea_ref, w2_ref, b_ref, s_ref):
    g = jnp.dot(ea_ref[...], w2_ref[...], preferred_element_type=jnp.float32)
    g = g + b_ref[...]
    s_ref[...] = g * jax.nn.sigmoid(g)


def _edge_gate(edge_attr, W2, b):
    return pl.pallas_call(
        _edge_gate_body,
        grid=(_E // _BE,),
        in_specs=[
            pl.BlockSpec((_BE, _DE), lambda j: (j, 0)),
            pl.BlockSpec((_DE, _D), lambda j: (0, 0)),
            pl.BlockSpec((1, _D), lambda j: (0, 0)),
        ],
        out_specs=pl.BlockSpec((_BE, _D), lambda j: (j, 0)),
        out_shape=jax.ShapeDtypeStruct((_E, _D), jnp.float32),
    )(edge_attr, W2, b.reshape(1, _D))


_NC = 2
_NS = 16
_TPW = _E // (_NC * _NS)
_CH = 80
_NCHUNK = _TPW // _CH
_RPAD = 10240
_RPT = _RPAD // _NS


def _segsum_body(s_hbm, idx_hbm, zeros_hbm, out_hbm, idx_v, rows_v, acc_sh):
    c = lax.axis_index("c")
    s = lax.axis_index("s")
    wid = c * _NS + s
    row0 = s * _RPT
    pltpu.sync_copy(zeros_hbm.at[pl.ds(row0, _RPT), :],
                    acc_sh.at[pl.ds(row0, _RPT), :])
    plsc.subcore_barrier()
    base = wid * _TPW

    def step(j, carry):
        off = pl.multiple_of(base + j * _CH, 8)
        pltpu.sync_copy(idx_hbm.at[pl.ds(off, _CH)], idx_v)
        pltpu.sync_copy(s_hbm.at[pl.ds(off, _CH), :], rows_v)
        pltpu.sync_copy(rows_v, acc_sh.at[idx_v], add=True)
        return carry

    lax.fori_loop(0, _NCHUNK, step, 0)
    plsc.subcore_barrier()
    pltpu.sync_copy(acc_sh.at[pl.ds(row0, _RPT), :],
                    out_hbm.at[c, pl.ds(row0, _RPT), :])


def _segsum(S, idx, zeros):
    mesh = plsc.VectorSubcoreMesh(core_axis_name="c", subcore_axis_name="s")
    fn = pl.kernel(
        _segsum_body,
        out_type=jax.ShapeDtypeStruct((_NC, _RPAD, _D), jnp.float32),
        mesh=mesh,
        scratch_types=[
            pltpu.VMEM((_CH,), jnp.int32),
            pltpu.VMEM((_CH, _D), jnp.float32),
            pltpu.VMEM_SHARED((_RPAD, _D), jnp.float32),
        ],
    )
    return fn(S, idx, zeros)


_BN = 2000


def _stats_body(x_ref, w1_ref, p_ref, t_ref, sum_ref, sq_ref):
    j = pl.program_id(0)
    t = jnp.dot(x_ref[...], w1_ref[...], preferred_element_type=jnp.float32)
    t = t * (p_ref[0] + p_ref[1])
    t_ref[...] = t

    @pl.when(j == 0)
    def _():
        sum_ref[...] = jnp.zeros_like(sum_ref)
        sq_ref[...] = jnp.zeros_like(sq_ref)

    sum_ref[...] += jnp.sum(t, axis=0, keepdims=True)
    sq_ref[...] += jnp.sum(t * t, axis=0, keepdims=True)


def _stats(x, W1, partials):
    return pl.pallas_call(
        _stats_body,
        grid=(_N // _BN,),
        in_specs=[
            pl.BlockSpec((_BN, _D), lambda j: (j, 0)),
            pl.BlockSpec((_D, _D), lambda j: (0, 0)),
            pl.BlockSpec((2, _BN, _D), lambda j: (0, j, 0)),
        ],
        out_specs=[
            pl.BlockSpec((_BN, _D), lambda j: (j, 0)),
            pl.BlockSpec((1, _D), lambda j: (0, 0)),
            pl.BlockSpec((1, _D), lambda j: (0, 0)),
        ],
        out_shape=[
            jax.ShapeDtypeStruct((_N, _D), jnp.float32),
            jax.ShapeDtypeStruct((1, _D), jnp.float32),
            jax.ShapeDtypeStruct((1, _D), jnp.float32),
        ],
    )(x, W1, partials)


def _norm_body(t_ref, x_ref, sum_ref, sq_ref, gamma_ref, beta_ref, o_ref):
    mean = sum_ref[...] / _N
    var = sq_ref[...] / _N - mean * mean
    rstd = lax.rsqrt(var + 1e-5)
    o_ref[...] = (t_ref[...] - mean) * rstd * gamma_ref[...] + beta_ref[...] + x_ref[...]


def _norm(t, x, ssum, ssq, gamma, beta):
    return pl.pallas_call(
        _norm_body,
        grid=(_N // _BN,),
        in_specs=[
            pl.BlockSpec((_BN, _D), lambda j: (j, 0)),
            pl.BlockSpec((_BN, _D), lambda j: (j, 0)),
            pl.BlockSpec((1, _D), lambda j: (0, 0)),
            pl.BlockSpec((1, _D), lambda j: (0, 0)),
            pl.BlockSpec((1, _D), lambda j: (0, 0)),
            pl.BlockSpec((1, _D), lambda j: (0, 0)),
        ],
        out_specs=pl.BlockSpec((_BN, _D), lambda j: (j, 0)),
        out_shape=jax.ShapeDtypeStruct((_N, _D), jnp.float32),
    )(t, x, ssum, ssq, gamma.reshape(1, _D), beta.reshape(1, _D))


def kernel(x, edge_index, edge_attr, W1, W2, b, gamma, beta):
    i = edge_index[0]
    S = _edge_gate(edge_attr, W2, b)
    zeros = jnp.zeros((_RPAD, _D), jnp.float32)
    partials = _segsum(S, i, zeros)
    t, ssum, ssq = _stats(x, W1, partials)
    return _norm(t, x, ssum, ssq, gamma, beta)


# speedup vs baseline: 6.0200x; 1.4414x over previous
import functools

import jax
import jax.numpy as jnp
from jax import lax
from jax.experimental import pallas as pl
from jax.experimental.pallas import tpu as pltpu
from jax.experimental.pallas import tpu_sc as plsc

_N = 10000
_E = 320000
_D = 128
_DE = 16

_BE = 8000


def _edge_gate_body(ea_ref, w2_ref, b_ref, s_ref):
    g = jnp.dot(ea_ref[...], w2_ref[...], preferred_element_type=jnp.float32)
    g = g + b_ref[...]
    s_ref[...] = g * jax.nn.sigmoid(g)


def _edge_gate(edge_attr, W2, b):
    return pl.pallas_call(
        _edge_gate_body,
        grid=(_E // _BE,),
        in_specs=[
            pl.BlockSpec((_BE, _DE), lambda j: (j, 0)),
            pl.BlockSpec((_DE, _D), lambda j: (0, 0)),
            pl.BlockSpec((1, _D), lambda j: (0, 0)),
        ],
        out_specs=pl.BlockSpec((_BE, _D), lambda j: (j, 0)),
        out_shape=jax.ShapeDtypeStruct((_E, _D), jnp.float32),
    )(edge_attr, W2, b.reshape(1, _D))


_NC = 2
_NS = 16
_TPW = _E // (_NC * _NS)
_CH = 128
_NFULL = _TPW // _CH
_TAIL = _TPW - _NFULL * _CH
_RPAD = 10240
_RPT = _RPAD // _NS


def _segsum_body(s_hbm, idx_hbm, zeros_hbm, out_hbm,
                 idx_v, rows_v, tidx_v, trow_v, sem0, sem1, acc_sh):
    c = lax.axis_index("c")
    s = lax.axis_index("s")
    wid = c * _NS + s
    row0 = s * _RPT
    pltpu.sync_copy(zeros_hbm.at[pl.ds(row0, _RPT), :],
                    acc_sh.at[pl.ds(row0, _RPT), :])
    plsc.subcore_barrier()
    base = wid * _TPW
    sems = (sem0, sem1)

    def start(j, b):
        off = pl.multiple_of(base + j * _CH, 8)
        pltpu.async_copy(idx_hbm.at[pl.ds(off, _CH)], idx_v.at[b], sems[b])
        pltpu.async_copy(s_hbm.at[pl.ds(off, _CH), :], rows_v.at[b], sems[b])

    def finish(b):
        pltpu.make_async_copy(idx_hbm.at[pl.ds(0, _CH)], idx_v.at[b], sems[b]).wait()
        pltpu.make_async_copy(s_hbm.at[pl.ds(0, _CH), :], rows_v.at[b], sems[b]).wait()

    start(0, 0)
    start(1, 1)

    def outer(g, carry):
        for b in range(2):
            finish(b)
            pltpu.sync_copy(rows_v.at[b], acc_sh.at[idx_v.at[b]], add=True)

            @pl.when(g <= _NFULL // 2 - 2)
            def _():
                start(g * 2 + b + 2, b)
        return carry

    lax.fori_loop(0, _NFULL // 2, outer, 0)
    toff = pl.multiple_of(base + _NFULL * _CH, 8)
    pltpu.sync_copy(idx_hbm.at[pl.ds(toff, _TAIL)], tidx_v)
    pltpu.sync_copy(s_hbm.at[pl.ds(toff, _TAIL), :], trow_v)
    pltpu.sync_copy(trow_v, acc_sh.at[tidx_v], add=True)
    plsc.subcore_barrier()
    pltpu.sync_copy(acc_sh.at[pl.ds(row0, _RPT), :],
                    out_hbm.at[c, pl.ds(row0, _RPT), :])


def _segsum(S, idx, zeros):
    mesh = plsc.VectorSubcoreMesh(core_axis_name="c", subcore_axis_name="s")
    fn = pl.kernel(
        _segsum_body,
        out_type=jax.ShapeDtypeStruct((_NC, _RPAD, _D), jnp.float32),
        mesh=mesh,
        scratch_types=[
            pltpu.VMEM((2, _CH), jnp.int32),
            pltpu.VMEM((2, _CH, _D), jnp.float32),
            pltpu.VMEM((_TAIL,), jnp.int32),
            pltpu.VMEM((_TAIL, _D), jnp.float32),
            pltpu.SemaphoreType.DMA,
            pltpu.SemaphoreType.DMA,
            pltpu.VMEM_SHARED((_RPAD, _D), jnp.float32),
        ],
    )
    return fn(S, idx, zeros)


_BN = 2000


def _stats_body(x_ref, w1_ref, p_ref, t_ref, sum_ref, sq_ref):
    j = pl.program_id(0)
    t = jnp.dot(x_ref[...], w1_ref[...], preferred_element_type=jnp.float32)
    t = t * (p_ref[0] + p_ref[1])
    t_ref[...] = t

    @pl.when(j == 0)
    def _():
        sum_ref[...] = jnp.zeros_like(sum_ref)
        sq_ref[...] = jnp.zeros_like(sq_ref)

    sum_ref[...] += jnp.sum(t, axis=0, keepdims=True)
    sq_ref[...] += jnp.sum(t * t, axis=0, keepdims=True)


def _stats(x, W1, partials):
    return pl.pallas_call(
        _stats_body,
        grid=(_N // _BN,),
        in_specs=[
            pl.BlockSpec((_BN, _D), lambda j: (j, 0)),
            pl.BlockSpec((_D, _D), lambda j: (0, 0)),
            pl.BlockSpec((2, _BN, _D), lambda j: (0, j, 0)),
        ],
        out_specs=[
            pl.BlockSpec((_BN, _D), lambda j: (j, 0)),
            pl.BlockSpec((1, _D), lambda j: (0, 0)),
            pl.BlockSpec((1, _D), lambda j: (0, 0)),
        ],
        out_shape=[
            jax.ShapeDtypeStruct((_N, _D), jnp.float32),
            jax.ShapeDtypeStruct((1, _D), jnp.float32),
            jax.ShapeDtypeStruct((1, _D), jnp.float32),
        ],
    )(x, W1, partials)


def _norm_body(t_ref, x_ref, sum_ref, sq_ref, gamma_ref, beta_ref, o_ref):
    mean = sum_ref[...] / _N
    var = sq_ref[...] / _N - mean * mean
    rstd = lax.rsqrt(var + 1e-5)
    o_ref[...] = (t_ref[...] - mean) * rstd * gamma_ref[...] + beta_ref[...] + x_ref[...]


def _norm(t, x, ssum, ssq, gamma, beta):
    return pl.pallas_call(
        _norm_body,
        grid=(_N // _BN,),
        in_specs=[
            pl.BlockSpec((_BN, _D), lambda j: (j, 0)),
            pl.BlockSpec((_BN, _D), lambda j: (j, 0)),
            pl.BlockSpec((1, _D), lambda j: (0, 0)),
            pl.BlockSpec((1, _D), lambda j: (0, 0)),
            pl.BlockSpec((1, _D), lambda j: (0, 0)),
            pl.BlockSpec((1, _D), lambda j: (0, 0)),
        ],
        out_specs=pl.BlockSpec((_BN, _D), lambda j: (j, 0)),
        out_shape=jax.ShapeDtypeStruct((_N, _D), jnp.float32),
    )(t, x, ssum, ssq, gamma.reshape(1, _D), beta.reshape(1, _D))


def kernel(x, edge_index, edge_attr, W1, W2, b, gamma, beta):
    i = edge_index[0]
    S = _edge_gate(edge_attr, W2, b)
    zeros = jnp.zeros((_RPAD, _D), jnp.float32)
    partials = _segsum(S, i, zeros)
    t, ssum, ssq = _stats(x, W1, partials)
    return _norm(t, x, ssum, ssq, gamma, beta)


# speedup vs baseline: 9.4828x; 1.5752x over previous
import functools

import jax
import jax.numpy as jnp
from jax import lax
from jax.experimental import pallas as pl
from jax.experimental.pallas import tpu as pltpu
from jax.experimental.pallas import tpu_sc as plsc

_N = 10000
_E = 320000
_D = 128
_DE = 16

_BE = 12800


def _edge_gate_body(eat_ref, w2_ref, b_ref, s_ref):
    g = jax.lax.dot_general(eat_ref[...], w2_ref[...],
                            dimension_numbers=(((0,), (0,)), ((), ())),
                            preferred_element_type=jnp.float32)
    g = g + b_ref[...]
    s_ref[...] = g * jax.nn.sigmoid(g)


def _edge_gate(edge_attr, W2, b):
    ea_t = edge_attr.T
    return pl.pallas_call(
        _edge_gate_body,
        grid=(_E // _BE,),
        in_specs=[
            pl.BlockSpec((_DE, _BE), lambda j: (0, j)),
            pl.BlockSpec((_DE, _D), lambda j: (0, 0)),
            pl.BlockSpec((1, _D), lambda j: (0, 0)),
        ],
        out_specs=pl.BlockSpec((_BE, _D), lambda j: (j, 0)),
        out_shape=jax.ShapeDtypeStruct((_E, _D), jnp.float32),
    )(ea_t, W2, b.reshape(1, _D))


_NC = 2
_NS = 16
_TPW = _E // (_NC * _NS)
_CH = 128
_NFULL = _TPW // _CH
_TAIL = _TPW - _NFULL * _CH
_RPAD = 10240
_RPT = _RPAD // _NS


def _segsum_body(s_hbm, idx_hbm, zeros_hbm, out_hbm,
                 idx_v, rows_v, tidx_v, trow_v, sem0, sem1, acc_sh):
    c = lax.axis_index("c")
    s = lax.axis_index("s")
    wid = c * _NS + s
    row0 = s * _RPT
    pltpu.sync_copy(zeros_hbm.at[pl.ds(row0, _RPT), :],
                    acc_sh.at[pl.ds(row0, _RPT), :])
    plsc.subcore_barrier()
    base = wid * _TPW
    sems = (sem0, sem1)

    def start(j, b):
        off = pl.multiple_of(base + j * _CH, 8)
        pltpu.async_copy(idx_hbm.at[pl.ds(off, _CH)], idx_v.at[b], sems[b])
        pltpu.async_copy(s_hbm.at[pl.ds(off, _CH), :], rows_v.at[b], sems[b])

    def finish(b):
        pltpu.make_async_copy(idx_hbm.at[pl.ds(0, _CH)], idx_v.at[b], sems[b]).wait()
        pltpu.make_async_copy(s_hbm.at[pl.ds(0, _CH), :], rows_v.at[b], sems[b]).wait()

    start(0, 0)
    start(1, 1)

    def outer(g, carry):
        for b in range(2):
            finish(b)
            pltpu.sync_copy(rows_v.at[b], acc_sh.at[idx_v.at[b]], add=True)

            @pl.when(g <= _NFULL // 2 - 2)
            def _():
                start(g * 2 + b + 2, b)
        return carry

    lax.fori_loop(0, _NFULL // 2, outer, 0)
    toff = pl.multiple_of(base + _NFULL * _CH, 8)
    pltpu.sync_copy(idx_hbm.at[pl.ds(toff, _TAIL)], tidx_v)
    pltpu.sync_copy(s_hbm.at[pl.ds(toff, _TAIL), :], trow_v)
    pltpu.sync_copy(trow_v, acc_sh.at[tidx_v], add=True)
    plsc.subcore_barrier()
    pltpu.sync_copy(acc_sh.at[pl.ds(row0, _RPT), :],
                    out_hbm.at[c, pl.ds(row0, _RPT), :])


def _segsum(S, idx, zeros):
    mesh = plsc.VectorSubcoreMesh(core_axis_name="c", subcore_axis_name="s")
    fn = pl.kernel(
        _segsum_body,
        out_type=jax.ShapeDtypeStruct((_NC, _RPAD, _D), jnp.float32),
        mesh=mesh,
        scratch_types=[
            pltpu.VMEM((2, _CH), jnp.int32),
            pltpu.VMEM((2, _CH, _D), jnp.float32),
            pltpu.VMEM((_TAIL,), jnp.int32),
            pltpu.VMEM((_TAIL, _D), jnp.float32),
            pltpu.SemaphoreType.DMA,
            pltpu.SemaphoreType.DMA,
            pltpu.VMEM_SHARED((_RPAD, _D), jnp.float32),
        ],
    )
    return fn(S, idx, zeros)


_BN = 2000


def _stats_body(x_ref, w1_ref, p_ref, t_ref, sum_ref, sq_ref):
    j = pl.program_id(0)
    t = jnp.dot(x_ref[...], w1_ref[...], preferred_element_type=jnp.float32)
    t = t * (p_ref[0] + p_ref[1])
    t_ref[...] = t

    @pl.when(j == 0)
    def _():
        sum_ref[...] = jnp.zeros_like(sum_ref)
        sq_ref[...] = jnp.zeros_like(sq_ref)

    sum_ref[...] += jnp.sum(t, axis=0, keepdims=True)
    sq_ref[...] += jnp.sum(t * t, axis=0, keepdims=True)


def _stats(x, W1, partials):
    return pl.pallas_call(
        _stats_body,
        grid=(_N // _BN,),
        in_specs=[
            pl.BlockSpec((_BN, _D), lambda j: (j, 0)),
            pl.BlockSpec((_D, _D), lambda j: (0, 0)),
            pl.BlockSpec((2, _BN, _D), lambda j: (0, j, 0)),
        ],
        out_specs=[
            pl.BlockSpec((_BN, _D), lambda j: (j, 0)),
            pl.BlockSpec((1, _D), lambda j: (0, 0)),
            pl.BlockSpec((1, _D), lambda j: (0, 0)),
        ],
        out_shape=[
            jax.ShapeDtypeStruct((_N, _D), jnp.float32),
            jax.ShapeDtypeStruct((1, _D), jnp.float32),
            jax.ShapeDtypeStruct((1, _D), jnp.float32),
        ],
    )(x, W1, partials)


def _norm_body(t_ref, x_ref, sum_ref, sq_ref, gamma_ref, beta_ref, o_ref):
    mean = sum_ref[...] / _N
    var = sq_ref[...] / _N - mean * mean
    rstd = lax.rsqrt(var + 1e-5)
    o_ref[...] = (t_ref[...] - mean) * rstd * gamma_ref[...] + beta_ref[...] + x_ref[...]


def _norm(t, x, ssum, ssq, gamma, beta):
    return pl.pallas_call(
        _norm_body,
        grid=(_N // _BN,),
        in_specs=[
            pl.BlockSpec((_BN, _D), lambda j: (j, 0)),
            pl.BlockSpec((_BN, _D), lambda j: (j, 0)),
            pl.BlockSpec((1, _D), lambda j: (0, 0)),
            pl.BlockSpec((1, _D), lambda j: (0, 0)),
            pl.BlockSpec((1, _D), lambda j: (0, 0)),
            pl.BlockSpec((1, _D), lambda j: (0, 0)),
        ],
        out_specs=pl.BlockSpec((_BN, _D), lambda j: (j, 0)),
        out_shape=jax.ShapeDtypeStruct((_N, _D), jnp.float32),
    )(t, x, ssum, ssq, gamma.reshape(1, _D), beta.reshape(1, _D))


def kernel(x, edge_index, edge_attr, W1, W2, b, gamma, beta):
    i = edge_index[0]
    S = _edge_gate(edge_attr, W2, b)
    zeros = jnp.zeros((_RPAD, _D), jnp.float32)
    partials = _segsum(S, i, zeros)
    t, ssum, ssq = _stats(x, W1, partials)
    return _norm(t, x, ssum, ssq, gamma, beta)


# speedup vs baseline: 10.4517x; 1.1022x over previous
import functools

import jax
import jax.numpy as jnp
from jax import lax
from jax.experimental import pallas as pl
from jax.experimental.pallas import tpu as pltpu
from jax.experimental.pallas import tpu_sc as plsc

_N = 10000
_E = 320000
_D = 128
_DE = 16

_BE = 12800


def _edge_gate_body(eat_ref, w2_ref, b_ref, s_ref):
    g = jax.lax.dot_general(eat_ref[...], w2_ref[...],
                            dimension_numbers=(((0,), (0,)), ((), ())),
                            preferred_element_type=jnp.float32)
    g = g + b_ref[...]
    s_ref[...] = g * jax.nn.sigmoid(g)


def _edge_gate(edge_attr, W2, b):
    ea_t = edge_attr.T
    return pl.pallas_call(
        _edge_gate_body,
        grid=(_E // _BE,),
        in_specs=[
            pl.BlockSpec((_DE, _BE), lambda j: (0, j)),
            pl.BlockSpec((_DE, _D), lambda j: (0, 0)),
            pl.BlockSpec((1, _D), lambda j: (0, 0)),
        ],
        out_specs=pl.BlockSpec((_BE, _D), lambda j: (j, 0)),
        out_shape=jax.ShapeDtypeStruct((_E, _D), jnp.float32),
    )(ea_t, W2, b.reshape(1, _D))


_NC = 2
_NS = 16
_CH = 128
_TOTCH = _E // _CH
_NCHT = _TOTCH // (_NC * _NS)
_XTRA = _TOTCH - _NCHT * _NC * _NS
_RPAD = 10112
_RPT = _RPAD // _NS


def _segsum_body(s_hbm, ei_hbm, out_hbm,
                 idx_v, rows_v, xidx_v, xrow_v, sem0, sem1, acc_sh):
    c = lax.axis_index("c")
    s = lax.axis_index("s")
    wid = c * _NS + s
    row0 = s * _RPT
    z16 = jnp.zeros((16,), jnp.float32)

    def zstep(r, carry):
        for k in range(_D // 16):
            rows_v[0, r, pl.ds(k * 16, 16)] = z16
        return carry

    lax.fori_loop(0, _CH, zstep, 0)
    for k in range(_RPT // _CH):
        pltpu.sync_copy(rows_v.at[0], acc_sh.at[pl.ds(row0 + k * _CH, _CH), :])
    _REM = _RPT - (_RPT // _CH) * _CH
    if _REM:
        pltpu.sync_copy(rows_v.at[0, pl.ds(0, _REM), :],
                        acc_sh.at[pl.ds(row0 + (_RPT // _CH) * _CH, _REM), :])
    plsc.subcore_barrier()

    sems = (sem0, sem1)
    cbase = wid * _NCHT

    def start(q, b):
        off = pl.multiple_of(q * _CH, _CH)
        pltpu.async_copy(ei_hbm.at[0, pl.ds(off, _CH)], idx_v.at[b], sems[b])
        pltpu.async_copy(s_hbm.at[pl.ds(off, _CH), :], rows_v.at[b], sems[b])

    def finish(b):
        pltpu.make_async_copy(ei_hbm.at[0, pl.ds(0, _CH)], idx_v.at[b], sems[b]).wait()
        pltpu.make_async_copy(s_hbm.at[pl.ds(0, _CH), :], rows_v.at[b], sems[b]).wait()

    start(cbase, 0)
    start(cbase + 1, 1)

    def outer(g, carry):
        for b in range(2):
            finish(b)
            pltpu.sync_copy(rows_v.at[b], acc_sh.at[idx_v.at[b]], add=True)

            @pl.when(g <= _NCHT // 2 - 2)
            def _():
                start(cbase + g * 2 + b + 2, b)
        return carry

    lax.fori_loop(0, _NCHT // 2, outer, 0)

    @pl.when(wid < _XTRA)
    def _():
        off = pl.multiple_of((_NCHT * _NC * _NS + wid) * _CH, _CH)
        pltpu.sync_copy(ei_hbm.at[0, pl.ds(off, _CH)], xidx_v)
        pltpu.sync_copy(s_hbm.at[pl.ds(off, _CH), :], xrow_v)
        pltpu.sync_copy(xrow_v, acc_sh.at[xidx_v], add=True)

    plsc.subcore_barrier()
    pltpu.sync_copy(acc_sh.at[pl.ds(row0, _RPT), :],
                    out_hbm.at[c, pl.ds(row0, _RPT), :])


def _segsum(S, edge_index):
    mesh = plsc.VectorSubcoreMesh(core_axis_name="c", subcore_axis_name="s")
    fn = pl.kernel(
        _segsum_body,
        out_type=jax.ShapeDtypeStruct((_NC, _RPAD, _D), jnp.float32),
        mesh=mesh,
        scratch_types=[
            pltpu.VMEM((2, _CH), jnp.int32),
            pltpu.VMEM((2, _CH, _D), jnp.float32),
            pltpu.VMEM((_CH,), jnp.int32),
            pltpu.VMEM((_CH, _D), jnp.float32),
            pltpu.SemaphoreType.DMA,
            pltpu.SemaphoreType.DMA,
            pltpu.VMEM_SHARED((_RPAD, _D), jnp.float32),
        ],
    )
    return fn(S, edge_index)


_BN = 2000


def _stats_body(x_ref, w1_ref, p_ref, t_ref, sum_ref, sq_ref):
    j = pl.program_id(0)
    t = jnp.dot(x_ref[...], w1_ref[...], preferred_element_type=jnp.float32)
    t = t * (p_ref[0] + p_ref[1])
    t_ref[...] = t

    @pl.when(j == 0)
    def _():
        sum_ref[...] = jnp.zeros_like(sum_ref)
        sq_ref[...] = jnp.zeros_like(sq_ref)

    sum_ref[...] += jnp.sum(t, axis=0, keepdims=True)
    sq_ref[...] += jnp.sum(t * t, axis=0, keepdims=True)


def _stats(x, W1, partials):
    return pl.pallas_call(
        _stats_body,
        grid=(_N // _BN,),
        in_specs=[
            pl.BlockSpec((_BN, _D), lambda j: (j, 0)),
            pl.BlockSpec((_D, _D), lambda j: (0, 0)),
            pl.BlockSpec((2, _BN, _D), lambda j: (0, j, 0)),
        ],
        out_specs=[
            pl.BlockSpec((_BN, _D), lambda j: (j, 0)),
            pl.BlockSpec((1, _D), lambda j: (0, 0)),
            pl.BlockSpec((1, _D), lambda j: (0, 0)),
        ],
        out_shape=[
            jax.ShapeDtypeStruct((_N, _D), jnp.float32),
            jax.ShapeDtypeStruct((1, _D), jnp.float32),
            jax.ShapeDtypeStruct((1, _D), jnp.float32),
        ],
    )(x, W1, partials)


def _norm_body(t_ref, x_ref, sum_ref, sq_ref, gamma_ref, beta_ref, o_ref):
    mean = sum_ref[...] / _N
    var = sq_ref[...] / _N - mean * mean
    rstd = lax.rsqrt(var + 1e-5)
    o_ref[...] = (t_ref[...] - mean) * rstd * gamma_ref[...] + beta_ref[...] + x_ref[...]


def _norm(t, x, ssum, ssq, gamma, beta):
    return pl.pallas_call(
        _norm_body,
        grid=(_N // _BN,),
        in_specs=[
            pl.BlockSpec((_BN, _D), lambda j: (j, 0)),
            pl.BlockSpec((_BN, _D), lambda j: (j, 0)),
            pl.BlockSpec((1, _D), lambda j: (0, 0)),
            pl.BlockSpec((1, _D), lambda j: (0, 0)),
            pl.BlockSpec((1, _D), lambda j: (0, 0)),
            pl.BlockSpec((1, _D), lambda j: (0, 0)),
        ],
        out_specs=pl.BlockSpec((_BN, _D), lambda j: (j, 0)),
        out_shape=jax.ShapeDtypeStruct((_N, _D), jnp.float32),
    )(t, x, ssum, ssq, gamma.reshape(1, _D), beta.reshape(1, _D))


def kernel(x, edge_index, edge_attr, W1, W2, b, gamma, beta):
    S = _edge_gate(edge_attr, W2, b)
    partials = _segsum(S, edge_index)
    t, ssum, ssq = _stats(x, W1, partials)
    return _norm(t, x, ssum, ssq, gamma, beta)
